# baseline (speedup 1.0000x reference)
import functools

import jax
import jax.numpy as jnp
from jax.experimental import pallas as pl
from jax.experimental.pallas import tpu as pltpu

_TOPK = 5
_INF = float("inf")


def _knn_block_kernel(f_ref, mb_ref, out_ref, slots_ref, *, nk, nq, bq, bk, k_valid):
    j = pl.program_id(0)
    i = pl.program_id(1)
    d = f_ref.shape[1]

    @pl.when((j == 0) & (i == 0))
    def _init():
        slots_ref[...] = jnp.full(slots_ref.shape, _INF, jnp.float32)

    f = f_ref[pl.ds(i * bq, bq), :]
    mb = mb_ref[...]
    fm = jax.lax.dot_general(
        f, mb, (((1,), (1,)), ((), ())), preferred_element_type=jnp.float32
    )
    m2 = jnp.sum(mb * mb, axis=1)
    col = j * bk + jax.lax.broadcasted_iota(jnp.int32, (bq, bk), 1)
    s = jnp.where(col < k_valid, m2[None, :] - 2.0 * fm, _INF)

    slots = slots_ref[pl.ds(i * bq, bq), :]
    thresh = jnp.max(slots, axis=1, keepdims=True)
    mn1 = jnp.min(s, axis=1, keepdims=True)
    any_need = jnp.min(mn1 - thresh) < 0.0

    @pl.when(any_need)
    def _extract():
        lane_bk = jax.lax.broadcasted_iota(jnp.int32, (bq, bk), 1)
        lane_s = jax.lax.broadcasted_iota(jnp.int32, (bq, _TOPK), 1)

        def round_fn(_, carry):
            s_c, slots_c = carry
            mn = jnp.min(s_c, axis=1, keepdims=True)
            idx = jnp.min(jnp.where(s_c == mn, lane_bk, bk), axis=1, keepdims=True)
            s_n = jnp.where(lane_bk == idx, _INF, s_c)
            mx = jnp.max(slots_c, axis=1, keepdims=True)
            amx = jnp.min(
                jnp.where(slots_c == mx, lane_s, _TOPK), axis=1, keepdims=True
            )
            take = (lane_s == amx) & (mn < mx)
            slots_n = jnp.where(take, jnp.broadcast_to(mn, slots_c.shape), slots_c)
            return s_n, slots_n

        _, new_slots = jax.lax.fori_loop(0, _TOPK, round_fn, (s, slots))
        slots_ref[pl.ds(i * bq, bq), :] = new_slots

    @pl.when(j == nk - 1)
    def _finish():
        f2 = jnp.sum(f * f, axis=1, keepdims=True)
        sl = slots_ref[pl.ds(i * bq, bq), :]
        total = jnp.sum(jnp.sqrt(jnp.maximum(f2 + sl, 1e-12)), axis=1, keepdims=True)
        out_ref[pl.ds(i * bq, bq), :] = total


def _run(features, memory_bank, block_q, block_k, interpret=False):
    q, d = features.shape
    k_rows = memory_bank.shape[0]
    nk = -(-k_rows // block_k)
    kp = nk * block_k
    if kp != k_rows:
        memory_bank = jnp.pad(memory_bank, ((0, kp - k_rows), (0, 0)))
    nq = q // block_q

    body = functools.partial(
        _knn_block_kernel, nk=nk, nq=nq, bq=block_q, bk=block_k, k_valid=k_rows
    )
    out = pl.pallas_call(
        body,
        grid=(nk, nq),
        in_specs=[
            pl.BlockSpec((q, d), lambda j, i: (0, 0)),
            pl.BlockSpec((block_k, d), lambda j, i: (j, 0)),
        ],
        out_specs=pl.BlockSpec((q, 1), lambda j, i: (0, 0)),
        out_shape=jax.ShapeDtypeStruct((q, 1), jnp.float32),
        scratch_shapes=[pltpu.VMEM((q, _TOPK), jnp.float32)],
        interpret=interpret,
    )(features, memory_bank)
    return out[:, 0]


def kernel(features, memory_bank, k):
    total = _run(features, memory_bank, block_q=256, block_k=256)
    return total / k


# speedup vs baseline: 61.5399x; 61.5399x over previous
import functools

import jax
import jax.numpy as jnp
from jax.experimental import pallas as pl
from jax.experimental.pallas import tpu as pltpu

_TOPK = 5
_LANES = 128
_INF = float("inf")


def _knn_kernel(f_ref, mb_ref, m2_ref, out_ref, *L_refs, nk, bk, cw):
    j = pl.program_id(0)
    q = f_ref.shape[0]

    @pl.when(j == 0)
    def _init():
        for r in L_refs:
            r[...] = jnp.full((q, cw), _INF, jnp.float32)

    f = f_ref[...]
    mb = mb_ref[...]
    fm = jax.lax.dot_general(
        f, mb, (((1,), (1,)), ((), ())), preferred_element_type=jnp.float32
    )
    s = m2_ref[0, :][None, :] - 2.0 * fm

    L = [r[...] for r in L_refs]
    for c in range(bk // cw):
        v = s[:, c * cw:(c + 1) * cw]
        for t in range(_TOPK):
            lo = jnp.minimum(L[t], v)
            v = jnp.maximum(L[t], v)
            L[t] = lo
    for r, val in zip(L_refs, L):
        r[...] = val

    @pl.when(j == nk - 1)
    def _finish():
        f2 = jnp.sum(f * f, axis=1, keepdims=True)
        cand = jnp.concatenate(L, axis=1)
        w = cand.shape[1]
        lane = jax.lax.broadcasted_iota(jnp.int32, (q, w), 1)
        total = jnp.zeros((q, 1), jnp.float32)
        for _ in range(_TOPK):
            mn = jnp.min(cand, axis=1, keepdims=True)
            idx = jnp.min(jnp.where(cand == mn, lane, w), axis=1, keepdims=True)
            cand = jnp.where(lane == idx, _INF, cand)
            total = total + jnp.sqrt(jnp.maximum(f2 + mn, 1e-12))
        out_ref[...] = total


def _run(features, memory_bank, block_k, interpret=False):
    q, d = features.shape
    k_rows = memory_bank.shape[0]
    nk = -(-k_rows // block_k)
    kp = nk * block_k
    if kp != k_rows:
        memory_bank = jnp.pad(memory_bank, ((0, kp - k_rows), (0, 0)))
    m2 = jnp.sum(memory_bank * memory_bank, axis=1)
    if kp != k_rows:
        m2 = m2.at[k_rows:].set(_INF)
    m2 = m2.reshape(1, kp)

    cw = min(_LANES, block_k)
    assert block_k % cw == 0
    body = functools.partial(_knn_kernel, nk=nk, bk=block_k, cw=cw)
    out = pl.pallas_call(
        body,
        grid=(nk,),
        in_specs=[
            pl.BlockSpec((q, d), lambda j: (0, 0)),
            pl.BlockSpec((block_k, d), lambda j: (j, 0)),
            pl.BlockSpec((1, block_k), lambda j: (0, j)),
        ],
        out_specs=pl.BlockSpec((q, 1), lambda j: (0, 0)),
        out_shape=jax.ShapeDtypeStruct((q, 1), jnp.float32),
        scratch_shapes=[pltpu.VMEM((q, cw), jnp.float32) for _ in range(_TOPK)],
        interpret=interpret,
    )(features, memory_bank, m2)
    return out[:, 0]


def kernel(features, memory_bank, k):
    total = _run(features, memory_bank, block_k=512)
    return total / k


# speedup vs baseline: 68.3968x; 1.1114x over previous
import functools

import jax
import jax.numpy as jnp
from jax.experimental import pallas as pl
from jax.experimental.pallas import tpu as pltpu

_TOPK = 5
_LANES = 128
_INF = float("inf")


def _knn_kernel(f_ref, mb_ref, m2_ref, out_ref, *L_refs, nk, bk, cw):
    j = pl.program_id(0)
    q = f_ref.shape[0]

    @pl.when(j == 0)
    def _init():
        for r in L_refs:
            r[...] = jnp.full((q, cw), _INF, jnp.float32)

    f = f_ref[...]
    mb = mb_ref[...]
    fm = jax.lax.dot_general(
        f, mb, (((1,), (1,)), ((), ())), preferred_element_type=jnp.float32
    )
    s = m2_ref[0, :][None, :] - 2.0 * fm

    L = [r[...] for r in L_refs]
    for c in range(bk // cw):
        v = s[:, c * cw:(c + 1) * cw]
        for t in range(_TOPK):
            lo = jnp.minimum(L[t], v)
            v = jnp.maximum(L[t], v)
            L[t] = lo
    for r, val in zip(L_refs, L):
        r[...] = val

    @pl.when(j == nk - 1)
    def _finish():
        f2 = jnp.sum(f * f, axis=1, keepdims=True)
        cand = jnp.concatenate(L, axis=1)
        w = cand.shape[1]
        lane = jax.lax.broadcasted_iota(jnp.int32, (q, w), 1)
        total = jnp.zeros((q, 1), jnp.float32)
        for _ in range(_TOPK):
            mn = jnp.min(cand, axis=1, keepdims=True)
            idx = jnp.min(jnp.where(cand == mn, lane, w), axis=1, keepdims=True)
            cand = jnp.where(lane == idx, _INF, cand)
            total = total + jnp.sqrt(jnp.maximum(f2 + mn, 1e-12))
        out_ref[...] = total


def _run(features, memory_bank, block_k, interpret=False):
    q, d = features.shape
    k_rows = memory_bank.shape[0]
    nk = -(-k_rows // block_k)
    kp = nk * block_k
    if kp != k_rows:
        memory_bank = jnp.pad(memory_bank, ((0, kp - k_rows), (0, 0)))
    m2 = jnp.sum(memory_bank * memory_bank, axis=1)
    if kp != k_rows:
        m2 = m2.at[k_rows:].set(_INF)
    m2 = m2.reshape(1, kp)

    cw = min(_LANES, block_k)
    assert block_k % cw == 0
    body = functools.partial(_knn_kernel, nk=nk, bk=block_k, cw=cw)
    out = pl.pallas_call(
        body,
        grid=(nk,),
        in_specs=[
            pl.BlockSpec((q, d), lambda j: (0, 0)),
            pl.BlockSpec((block_k, d), lambda j: (j, 0)),
            pl.BlockSpec((1, block_k), lambda j: (0, j)),
        ],
        out_specs=pl.BlockSpec((q, 1), lambda j: (0, 0)),
        out_shape=jax.ShapeDtypeStruct((q, 1), jnp.float32),
        scratch_shapes=[pltpu.VMEM((q, cw), jnp.float32) for _ in range(_TOPK)],
        interpret=interpret,
    )(features, memory_bank, m2)
    return out[:, 0]


def kernel(features, memory_bank, k):
    total = _run(features, memory_bank, block_k=1024)
    return total / k
